# baseline (speedup 1.0000x reference)
import functools

import jax
import jax.numpy as jnp
from jax import lax
from jax.experimental import pallas as pl
from jax.experimental.pallas import tpu as pltpu
from jax.experimental.pallas import tpu_sc as plsc

_NW = 32
_NBUF = 3


def _gather_kernel(n_rows, d, chunk):
    b_per_w = n_rows // _NW
    n_chunks = b_per_w // chunk
    mesh = plsc.VectorSubcoreMesh(core_axis_name="c", subcore_axis_name="s")

    @functools.partial(
        pl.kernel,
        mesh=mesh,
        out_type=jax.ShapeDtypeStruct((n_rows, d), jnp.float32),
        scratch_types=[
            pltpu.VMEM((b_per_w,), jnp.int32),
            [pltpu.VMEM((chunk, d), jnp.float32) for _ in range(_NBUF)],
            [pltpu.SemaphoreType.DMA for _ in range(_NBUF)],
            [pltpu.SemaphoreType.DMA for _ in range(_NBUF)],
        ],
        compiler_params=pltpu.CompilerParams(use_tc_tiling_on_sc=False),
    )
    def k(idx_hbm, table_hbm, out_hbm, idx_all, rows, sem_g, sem_o):
        cid = lax.axis_index("c")
        sid = lax.axis_index("s")
        wid = sid * 2 + cid
        base = wid * b_per_w

        pltpu.sync_copy(idx_hbm.at[pl.ds(base, b_per_w)], idx_all)

        def scale(m, _):
            idx_all[pl.ds(m * 16, 16)] = idx_all[pl.ds(m * 16, 16)] * 4
            return 0

        lax.fori_loop(0, b_per_w // 16, scale, 0)

        gathers = {}
        stores = {}

        def start_store(j):
            r = j % _NBUF
            gathers[j].wait()
            stores[j] = pltpu.async_copy(
                rows[r], out_hbm.at[pl.ds(base + j * chunk, chunk)], sem_o[r]
            )

        for i in range(n_chunks):
            r = i % _NBUF
            if i >= _NBUF:
                stores[i - _NBUF].wait()
            gathers[i] = pltpu.async_copy(
                table_hbm.at[idx_all.at[pl.ds(i * chunk, chunk)]], rows[r], sem_g[r]
            )
            if i >= 1:
                start_store(i - 1)
        start_store(n_chunks - 1)
        for j in range(max(0, n_chunks - _NBUF + 1), n_chunks):
            stores[j].wait()

    return k


def kernel(x, table):
    b, f = x.shape
    v, d = table.shape
    n_rows = b * f
    chunk = 1024
    idx = x.reshape(n_rows)
    t4 = jnp.repeat(table, 4, axis=0)
    out = _gather_kernel(n_rows, d, chunk)(idx, t4)
    return out.reshape(b, f * d)


# speedup vs baseline: 3.8289x; 3.8289x over previous
import functools

import jax
import jax.numpy as jnp
from jax import lax
from jax.experimental import pallas as pl
from jax.experimental.pallas import tpu as pltpu
from jax.experimental.pallas import tpu_sc as plsc

_NW = 32
_NBUF = 3


def _gather_kernel(n_rows, d, chunk):
    b_per_w = n_rows // _NW
    n_chunks = b_per_w // chunk
    mesh = plsc.VectorSubcoreMesh(core_axis_name="c", subcore_axis_name="s")

    @functools.partial(
        pl.kernel,
        mesh=mesh,
        out_type=jax.ShapeDtypeStruct((n_rows, d), jnp.float32),
        scratch_types=[
            pltpu.VMEM((b_per_w,), jnp.int32),
            [pltpu.VMEM((chunk, d), jnp.float32) for _ in range(_NBUF)],
            [pltpu.SemaphoreType.DMA for _ in range(_NBUF)],
            [pltpu.SemaphoreType.DMA for _ in range(_NBUF)],
        ],
        compiler_params=pltpu.CompilerParams(use_tc_tiling_on_sc=False),
    )
    def k(idx_hbm, table_hbm, out_hbm, idx_all, rows, sem_g, sem_o):
        cid = lax.axis_index("c")
        sid = lax.axis_index("s")
        wid = sid * 2 + cid
        base = wid * b_per_w

        pltpu.sync_copy(idx_hbm.at[pl.ds(base, b_per_w)], idx_all)

        gathers = {}
        stores = {}

        def start_store(j):
            r = j % _NBUF
            gathers[j].wait()
            stores[j] = pltpu.async_copy(
                rows[r], out_hbm.at[pl.ds(base + j * chunk, chunk)], sem_o[r]
            )

        for i in range(n_chunks):
            r = i % _NBUF
            if i >= _NBUF:
                stores[i - _NBUF].wait()
            gathers[i] = pltpu.async_copy(
                table_hbm.at[idx_all.at[pl.ds(i * chunk, chunk)]], rows[r], sem_g[r]
            )
            if i >= 1:
                start_store(i - 1)
        start_store(n_chunks - 1)
        for j in range(max(0, n_chunks - _NBUF + 1), n_chunks):
            stores[j].wait()

    return k


def kernel(x, table):
    b, f = x.shape
    v, d = table.shape
    n_rows = b * f
    chunk = 1024
    idx = x.reshape(n_rows)
    out = _gather_kernel(n_rows, d, chunk)(idx, table)
    return out.reshape(b, f * d)
